# Initial kernel scaffold; baseline (speedup 1.0000x reference)
import jax
import jax.numpy as jnp
from jax.experimental import pallas as pl


def kernel(node_features, edge_weights, lstm_kernel, lstm_recurrent, lstm_bias, c1_prep_W, c1_prep_b, c1_upd_W, c1_upd_b, c2_prep_W, c2_prep_b, c2_upd_W, c2_upd_b, post_W, post_b, logits_W, logits_b, edges, input_node_indices):
    raise NotImplementedError("write your pallas kernel here")


# speedup vs baseline: 4.9902x; 4.9902x over previous
import math

import jax
import jax.numpy as jnp
from jax import lax
from jax.experimental import pallas as pl
from jax.experimental.pallas import tpu as pltpu
from jax.experimental.pallas import tpu_sc as plsc

N_NODES = 50000
N_EDGES = 800000
T = 8
F = 64
U = 64
H = 64
NUM_CLASSES = 40
BATCH = 1024
BN_INV = (1.0 + 1e-3) ** -0.5

NC = 2
NS = 16
QW = 16
NQ = 4
PASSES = NQ // NC

CHUNK = 1024
SUB = CHUNK // 128
CHUNKS_PER_TILE = math.ceil(N_EDGES / (NS * CHUNK))
EDGES_PAD = NS * CHUNK * CHUNKS_PER_TILE
NP = 51200
ROWS_PER_TILE = NP // NS
ZCH = 5
ZROWS = ROWS_PER_TILE // ZCH


def _sum_body(x_ref, o_ref):
    o_ref[0, 0] = jnp.sum(x_ref[...])


def _ew_sum(ew_pad2d):
    return pl.pallas_call(
        _sum_body,
        out_specs=pl.BlockSpec(memory_space=pltpu.SMEM),
        out_shape=jax.ShapeDtypeStruct((1, 1), jnp.float32),
    )(ew_pad2d)


def _gelu(x):
    return 0.5 * x * (1.0 + lax.erf(x * (2.0 ** -0.5)))


def _lstm_body(x_ref, wc_ref, b_ref, wp_ref, bp_ref, x1_ref, y1_ref):
    xb = x_ref[...]
    bn = xb.shape[0]
    wc = wc_ref[...]
    b = b_ref[...]
    h = jnp.zeros((bn, U), jnp.float32)
    c = jnp.zeros((bn, U), jnp.float32)
    for t in range(T):
        xt = xb[:, t, :]
        z = jnp.dot(jnp.concatenate([xt, h], axis=1), wc,
                    preferred_element_type=jnp.float32) + b
        zi = z[:, 0:U]
        zf = z[:, U:2 * U]
        zg = z[:, 2 * U:3 * U]
        zo = z[:, 3 * U:4 * U]
        c = jax.nn.sigmoid(zf) * c + jax.nn.sigmoid(zi) * jnp.tanh(zg)
        h = jax.nn.sigmoid(zo) * jnp.tanh(c)
    x1_ref[...] = h
    y = _gelu(jnp.dot(h, wp_ref[...], preferred_element_type=jnp.float32)
              + bp_ref[...])
    for q in range(NQ):
        y1_ref[q] = y[:, q * QW:(q + 1) * QW]


def _lstm_prep(nf, wcat, b4, wp, bp, bn=2000):
    grid = N_NODES // bn
    return pl.pallas_call(
        _lstm_body,
        grid=(grid,),
        in_specs=[
            pl.BlockSpec((bn, T, F), lambda i: (i, 0, 0)),
            pl.BlockSpec((F + U, 4 * U), lambda i: (0, 0)),
            pl.BlockSpec((1, 4 * U), lambda i: (0, 0)),
            pl.BlockSpec((U, H), lambda i: (0, 0)),
            pl.BlockSpec((1, H), lambda i: (0, 0)),
        ],
        out_specs=[
            pl.BlockSpec((bn, U), lambda i: (i, 0)),
            pl.BlockSpec((NQ, bn, QW), lambda i: (0, i, 0)),
        ],
        out_shape=[
            jax.ShapeDtypeStruct((N_NODES, U), jnp.float32),
            jax.ShapeDtypeStruct((NQ, N_NODES, QW), jnp.float32),
        ],
    )(nf, wcat, b4, wp, bp)


def _upd_body(x_ref, agg_ref, s_ref, wx_ref, wa_ref, bu_ref,
              wp_ref, bp_ref, x2_ref, y2_ref):
    x = x_ref[...]
    inv_s = 1.0 / s_ref[0, 0]
    a = jnp.dot(agg_ref[0], wa_ref[0], preferred_element_type=jnp.float32)
    for q in range(1, NQ):
        a = a + jnp.dot(agg_ref[q], wa_ref[q], preferred_element_type=jnp.float32)
    u = (jnp.dot(x, wx_ref[...], preferred_element_type=jnp.float32)
         + a * inv_s + bu_ref[...])
    o = _gelu(u)
    o = o * lax.rsqrt(jnp.maximum(jnp.sum(o * o, axis=-1, keepdims=True), 1e-12))
    x2_ref[...] = o
    y = _gelu(jnp.dot(o, wp_ref[...], preferred_element_type=jnp.float32)
              + bp_ref[...])
    for q in range(NQ):
        y2_ref[q] = y[:, q * QW:(q + 1) * QW]


def _upd_prep(x1, agg, s, wx, wa, bu, wp, bp, bn=2000):
    grid = N_NODES // bn
    return pl.pallas_call(
        _upd_body,
        grid=(grid,),
        in_specs=[
            pl.BlockSpec((bn, U), lambda i: (i, 0)),
            pl.BlockSpec((NQ, bn, QW), lambda i: (0, i, 0)),
            pl.BlockSpec(memory_space=pltpu.SMEM),
            pl.BlockSpec((U, H), lambda i: (0, 0)),
            pl.BlockSpec((NQ, QW, H), lambda i: (0, 0, 0)),
            pl.BlockSpec((1, H), lambda i: (0, 0)),
            pl.BlockSpec((H, H), lambda i: (0, 0)),
            pl.BlockSpec((1, H), lambda i: (0, 0)),
        ],
        out_specs=[
            pl.BlockSpec((bn, H), lambda i: (i, 0)),
            pl.BlockSpec((NQ, bn, QW), lambda i: (0, i, 0)),
        ],
        out_shape=[
            jax.ShapeDtypeStruct((N_NODES, H), jnp.float32),
            jax.ShapeDtypeStruct((NQ, N_NODES, QW), jnp.float32),
        ],
    )(x1, agg, s, wx, wa, bu, wp, bp)


def _head_body(xg_ref, ag_ref, s_ref, wx_ref, wa_ref,
               bu_ref, wpo_ref, bpo_ref, wlo_ref, blo_ref, o_ref):
    inv_s = 1.0 / s_ref[0, 0]
    a = jnp.dot(ag_ref[0], wa_ref[0], preferred_element_type=jnp.float32)
    for q in range(1, NQ):
        a = a + jnp.dot(ag_ref[q], wa_ref[q], preferred_element_type=jnp.float32)
    u = (jnp.dot(xg_ref[...], wx_ref[...], preferred_element_type=jnp.float32)
         + a * inv_s + bu_ref[...])
    o = _gelu(u)
    x3 = o * lax.rsqrt(jnp.maximum(jnp.sum(o * o, axis=-1, keepdims=True), 1e-12))
    x4 = _gelu(jnp.dot(x3, wpo_ref[...], preferred_element_type=jnp.float32)
               + bpo_ref[...])
    o_ref[...] = (jnp.dot(x4, wlo_ref[...], preferred_element_type=jnp.float32)
                  + blo_ref[...])


def _head(xg, ag, s, wx, wa, bu, wpo, bpo, wlo_pad, blo_pad):
    return pl.pallas_call(
        _head_body,
        in_specs=[
            pl.BlockSpec((BATCH, H), lambda: (0, 0)),
            pl.BlockSpec((NQ, BATCH, QW), lambda: (0, 0, 0)),
            pl.BlockSpec(memory_space=pltpu.SMEM),
            pl.BlockSpec((H, H), lambda: (0, 0)),
            pl.BlockSpec((NQ, QW, H), lambda: (0, 0, 0)),
            pl.BlockSpec((1, H), lambda: (0, 0)),
            pl.BlockSpec((H, H), lambda: (0, 0)),
            pl.BlockSpec((1, H), lambda: (0, 0)),
            pl.BlockSpec((H, 128), lambda: (0, 0)),
            pl.BlockSpec((1, 128), lambda: (0, 0)),
        ],
        out_specs=pl.BlockSpec((BATCH, 128), lambda: (0, 0)),
        out_shape=jax.ShapeDtypeStruct((BATCH, 128), jnp.float32),
    )(xg, ag, s, wx, wa, bu, wpo, bpo, wlo_pad, blo_pad)


def _sc_scatter_body(y2d, srcr, dstr, ewr, out,
                     src_v, dst_v, ew_v, rows_v, zbuf, spmem, sem):
    c = lax.axis_index("c")
    s = lax.axis_index("s")

    def zfill(i, _):
        zbuf[i, 0:16] = jnp.zeros((16,), jnp.float32)
        return 0
    lax.fori_loop(0, ZROWS, zfill, 0)

    for p in range(PASSES):
        q = c * PASSES + p
        src_off = q * N_NODES

        for r in range(ZCH):
            pltpu.sync_copy(
                zbuf, spmem.at[pl.ds(s * ROWS_PER_TILE + r * ZROWS, ZROWS)])
        plsc.subcore_barrier()

        def chunk(k, _):
            base = (s * CHUNKS_PER_TILE + k) * SUB
            pltpu.sync_copy(srcr.at[pl.ds(base, SUB)], src_v)
            pltpu.sync_copy(dstr.at[pl.ds(base, SUB)], dst_v)
            pltpu.sync_copy(ewr.at[pl.ds(base, SUB)], ew_v)
            for j in range(SUB):
                for l in range(8):
                    src_v[j, l * 16:(l + 1) * 16] = (
                        src_v[j, l * 16:(l + 1) * 16] + src_off)
            cps = [pltpu.async_copy(y2d.at[src_v.at[j]],
                                    rows_v.at[pl.ds(j * 128, 128)], sem)
                   for j in range(SUB)]
            for cp in cps:
                cp.wait()

            def scale16(g, _):
                ewv = ew_v[g // 8, pl.ds((g % 8) * 16, 16)]
                r0 = g * 16
                for t in range(16):
                    w = ewv[t]
                    rows_v[r0 + t, 0:16] = rows_v[r0 + t, 0:16] * w
                return 0
            lax.fori_loop(0, CHUNK // 16, scale16, 0)

            for j in range(SUB):
                pltpu.sync_copy(rows_v.at[pl.ds(j * 128, 128)],
                                spmem.at[dst_v.at[j]], add=True)
            return 0

        lax.fori_loop(0, CHUNKS_PER_TILE, chunk, 0)
        plsc.subcore_barrier()

        for r in range(ZCH):
            row0 = s * ROWS_PER_TILE + r * ZROWS
            pltpu.sync_copy(spmem.at[pl.ds(row0, ZROWS)],
                            out.at[pl.ds(q * NP + row0, ZROWS)])


def _sc_scatter(y2d, srcr, dstr, ewr):
    mesh = plsc.VectorSubcoreMesh(core_axis_name="c", subcore_axis_name="s",
                                  num_cores=NC, num_subcores=NS)
    f = pl.kernel(
        _sc_scatter_body,
        out_type=jax.ShapeDtypeStruct((NQ * NP, QW), jnp.float32),
        mesh=mesh,
        compiler_params=pltpu.CompilerParams(use_tc_tiling_on_sc=False),
        scratch_types=[
            pltpu.VMEM((SUB, 128), jnp.int32),
            pltpu.VMEM((SUB, 128), jnp.int32),
            pltpu.VMEM((SUB, 128), jnp.float32),
            pltpu.VMEM((CHUNK, QW), jnp.float32),
            pltpu.VMEM((ZROWS, QW), jnp.float32),
            pltpu.VMEM_SHARED((NP, QW), jnp.float32),
            pltpu.SemaphoreType.DMA,
        ],
    )
    return f(y2d, srcr, dstr, ewr)


def _sc_gather_body(x2, aggf, idxr, xg, ag, idx_v, idx2_v, rows64, rows16, sem):
    c = lax.axis_index("c")
    s = lax.axis_index("s")
    wid = c * NS + s
    bper = BATCH // (NC * NS)
    base = wid * bper
    pltpu.sync_copy(idxr.at[pl.ds(base, bper)], idx_v)
    pltpu.async_copy(x2.at[idx_v], rows64, sem).wait()
    pltpu.sync_copy(rows64, xg.at[pl.ds(base, bper)])
    for q in range(NQ):
        for l in range(bper // 16):
            idx2_v[l * 16:(l + 1) * 16] = (idx_v[l * 16:(l + 1) * 16]
                                           + q * NP)
        pltpu.async_copy(aggf.at[idx2_v], rows16, sem).wait()
        pltpu.sync_copy(rows16, ag.at[q, pl.ds(base, bper)])


def _sc_gather(x2, aggf, idx):
    bper = BATCH // (NC * NS)
    mesh = plsc.VectorSubcoreMesh(core_axis_name="c", subcore_axis_name="s",
                                  num_cores=NC, num_subcores=NS)
    f = pl.kernel(
        _sc_gather_body,
        compiler_params=pltpu.CompilerParams(use_tc_tiling_on_sc=False),
        out_type=[
            jax.ShapeDtypeStruct((BATCH, H), jnp.float32),
            jax.ShapeDtypeStruct((NQ, BATCH, QW), jnp.float32),
        ],
        mesh=mesh,
        scratch_types=[
            pltpu.VMEM((bper,), jnp.int32),
            pltpu.VMEM((bper,), jnp.int32),
            pltpu.VMEM((bper, H), jnp.float32),
            pltpu.VMEM((bper, QW), jnp.float32),
            pltpu.SemaphoreType.DMA,
        ],
    )
    return f(x2, aggf, idx)


def kernel(node_features, edge_weights, lstm_kernel, lstm_recurrent, lstm_bias,
           c1_prep_W, c1_prep_b, c1_upd_W, c1_upd_b,
           c2_prep_W, c2_prep_b, c2_upd_W, c2_upd_b,
           post_W, post_b, logits_W, logits_b,
           edges, input_node_indices):
    pad = EDGES_PAD - N_EDGES
    src = jnp.pad(edges[1], (0, pad)).reshape(EDGES_PAD // 128, 128)
    dst = jnp.pad(edges[0], (0, pad)).reshape(EDGES_PAD // 128, 128)
    ewp = jnp.pad(edge_weights, (0, pad)).reshape(EDGES_PAD // 128, 128)

    wcat = jnp.concatenate([lstm_kernel, lstm_recurrent], axis=0)
    b4 = lstm_bias.reshape(1, 4 * U)
    wp1 = c1_prep_W * BN_INV
    bp1 = c1_prep_b.reshape(1, H)
    u1 = c1_upd_W * BN_INV
    wx1 = u1[:U]
    wa1 = u1[U:].reshape(NQ, QW, H)
    bu1 = c1_upd_b.reshape(1, H)
    wp2 = c2_prep_W * BN_INV
    bp2 = c2_prep_b.reshape(1, H)
    u2 = c2_upd_W * BN_INV
    wx2 = u2[:H]
    wa2 = u2[H:].reshape(NQ, QW, H)
    bu2 = c2_upd_b.reshape(1, H)
    wpo = post_W * BN_INV
    bpo = post_b.reshape(1, H)
    wlo = jnp.pad(logits_W, ((0, 0), (0, 128 - NUM_CLASSES)))
    blo = jnp.pad(logits_b, (0, 128 - NUM_CLASSES)).reshape(1, 128)

    s_sum = _ew_sum(ewp)
    x1, y1 = _lstm_prep(node_features, wcat, b4, wp1, bp1)
    agg1 = _sc_scatter(y1.reshape(NQ * N_NODES, QW), src, dst, ewp)
    x2, y2 = _upd_prep(x1, agg1.reshape(NQ, NP, QW), s_sum,
                       wx1, wa1, bu1, wp2, bp2)
    agg2 = _sc_scatter(y2.reshape(NQ * N_NODES, QW), src, dst, ewp)
    xg, ag = _sc_gather(x2, agg2, input_node_indices)
    out = _head(xg, ag, s_sum, wx2, wa2, bu2, wpo, bpo, wlo, blo)
    return out[:, :NUM_CLASSES]


# speedup vs baseline: 5.8661x; 1.1755x over previous
import math

import jax
import jax.numpy as jnp
from jax import lax
from jax.experimental import pallas as pl
from jax.experimental.pallas import tpu as pltpu
from jax.experimental.pallas import tpu_sc as plsc

N_NODES = 50000
N_EDGES = 800000
T = 8
F = 64
U = 64
H = 64
NUM_CLASSES = 40
BATCH = 1024
BN_INV = (1.0 + 1e-3) ** -0.5

NC = 2
NS = 16
QW = 16
NQ = 4
PASSES = NQ // NC

CHUNK = 1024
SUB = CHUNK // 128
CHUNKS_PER_TILE = 2 * math.ceil(N_EDGES / (NS * CHUNK * 2))
EDGES_PAD = NS * CHUNK * CHUNKS_PER_TILE
NP = 51200
ROWS_PER_TILE = NP // NS
ZCH = 5
ZROWS = ROWS_PER_TILE // ZCH


def _sum_body(x_ref, o_ref):
    o_ref[0, 0] = jnp.sum(x_ref[...])


def _ew_sum(ew_pad2d):
    return pl.pallas_call(
        _sum_body,
        out_specs=pl.BlockSpec(memory_space=pltpu.SMEM),
        out_shape=jax.ShapeDtypeStruct((1, 1), jnp.float32),
    )(ew_pad2d)


def _gelu(x):
    return 0.5 * x * (1.0 + lax.erf(x * (2.0 ** -0.5)))


def _lstm_body(x_ref, wc_ref, b_ref, wp_ref, bp_ref, x1_ref, y1_ref):
    xb = x_ref[...]
    bn = xb.shape[0]
    wc = wc_ref[...]
    b = b_ref[...]
    h = jnp.zeros((bn, U), jnp.float32)
    c = jnp.zeros((bn, U), jnp.float32)
    for t in range(T):
        xt = xb[:, t, :]
        z = jnp.dot(jnp.concatenate([xt, h], axis=1), wc,
                    preferred_element_type=jnp.float32) + b
        zi = z[:, 0:U]
        zf = z[:, U:2 * U]
        zg = z[:, 2 * U:3 * U]
        zo = z[:, 3 * U:4 * U]
        c = jax.nn.sigmoid(zf) * c + jax.nn.sigmoid(zi) * jnp.tanh(zg)
        h = jax.nn.sigmoid(zo) * jnp.tanh(c)
    x1_ref[...] = h
    y = _gelu(jnp.dot(h, wp_ref[...], preferred_element_type=jnp.float32)
              + bp_ref[...])
    for q in range(NQ):
        y1_ref[q] = y[:, q * QW:(q + 1) * QW]


def _lstm_prep(nf, wcat, b4, wp, bp, bn=2000):
    grid = N_NODES // bn
    return pl.pallas_call(
        _lstm_body,
        grid=(grid,),
        in_specs=[
            pl.BlockSpec((bn, T, F), lambda i: (i, 0, 0)),
            pl.BlockSpec((F + U, 4 * U), lambda i: (0, 0)),
            pl.BlockSpec((1, 4 * U), lambda i: (0, 0)),
            pl.BlockSpec((U, H), lambda i: (0, 0)),
            pl.BlockSpec((1, H), lambda i: (0, 0)),
        ],
        out_specs=[
            pl.BlockSpec((bn, U), lambda i: (i, 0)),
            pl.BlockSpec((NQ, bn, QW), lambda i: (0, i, 0)),
        ],
        out_shape=[
            jax.ShapeDtypeStruct((N_NODES, U), jnp.float32),
            jax.ShapeDtypeStruct((NQ, N_NODES, QW), jnp.float32),
        ],
    )(nf, wcat, b4, wp, bp)


def _upd_body(x_ref, agg_ref, s_ref, wx_ref, wa_ref, bu_ref,
              wp_ref, bp_ref, x2_ref, y2_ref):
    x = x_ref[...]
    inv_s = 1.0 / s_ref[0, 0]
    a = jnp.dot(agg_ref[0], wa_ref[0], preferred_element_type=jnp.float32)
    for q in range(1, NQ):
        a = a + jnp.dot(agg_ref[q], wa_ref[q], preferred_element_type=jnp.float32)
    u = (jnp.dot(x, wx_ref[...], preferred_element_type=jnp.float32)
         + a * inv_s + bu_ref[...])
    o = _gelu(u)
    o = o * lax.rsqrt(jnp.maximum(jnp.sum(o * o, axis=-1, keepdims=True), 1e-12))
    x2_ref[...] = o
    y = _gelu(jnp.dot(o, wp_ref[...], preferred_element_type=jnp.float32)
              + bp_ref[...])
    for q in range(NQ):
        y2_ref[q] = y[:, q * QW:(q + 1) * QW]


def _upd_prep(x1, agg, s, wx, wa, bu, wp, bp, bn=2000):
    grid = N_NODES // bn
    return pl.pallas_call(
        _upd_body,
        grid=(grid,),
        in_specs=[
            pl.BlockSpec((bn, U), lambda i: (i, 0)),
            pl.BlockSpec((NQ, bn, QW), lambda i: (0, i, 0)),
            pl.BlockSpec(memory_space=pltpu.SMEM),
            pl.BlockSpec((U, H), lambda i: (0, 0)),
            pl.BlockSpec((NQ, QW, H), lambda i: (0, 0, 0)),
            pl.BlockSpec((1, H), lambda i: (0, 0)),
            pl.BlockSpec((H, H), lambda i: (0, 0)),
            pl.BlockSpec((1, H), lambda i: (0, 0)),
        ],
        out_specs=[
            pl.BlockSpec((bn, H), lambda i: (i, 0)),
            pl.BlockSpec((NQ, bn, QW), lambda i: (0, i, 0)),
        ],
        out_shape=[
            jax.ShapeDtypeStruct((N_NODES, H), jnp.float32),
            jax.ShapeDtypeStruct((NQ, N_NODES, QW), jnp.float32),
        ],
    )(x1, agg, s, wx, wa, bu, wp, bp)


def _head_body(xg_ref, ag_ref, s_ref, wx_ref, wa_ref,
               bu_ref, wpo_ref, bpo_ref, wlo_ref, blo_ref, o_ref):
    inv_s = 1.0 / s_ref[0, 0]
    a = jnp.dot(ag_ref[0], wa_ref[0], preferred_element_type=jnp.float32)
    for q in range(1, NQ):
        a = a + jnp.dot(ag_ref[q], wa_ref[q], preferred_element_type=jnp.float32)
    u = (jnp.dot(xg_ref[...], wx_ref[...], preferred_element_type=jnp.float32)
         + a * inv_s + bu_ref[...])
    o = _gelu(u)
    x3 = o * lax.rsqrt(jnp.maximum(jnp.sum(o * o, axis=-1, keepdims=True), 1e-12))
    x4 = _gelu(jnp.dot(x3, wpo_ref[...], preferred_element_type=jnp.float32)
               + bpo_ref[...])
    o_ref[...] = (jnp.dot(x4, wlo_ref[...], preferred_element_type=jnp.float32)
                  + blo_ref[...])


def _head(xg, ag, s, wx, wa, bu, wpo, bpo, wlo_pad, blo_pad):
    return pl.pallas_call(
        _head_body,
        in_specs=[
            pl.BlockSpec((BATCH, H), lambda: (0, 0)),
            pl.BlockSpec((NQ, BATCH, QW), lambda: (0, 0, 0)),
            pl.BlockSpec(memory_space=pltpu.SMEM),
            pl.BlockSpec((H, H), lambda: (0, 0)),
            pl.BlockSpec((NQ, QW, H), lambda: (0, 0, 0)),
            pl.BlockSpec((1, H), lambda: (0, 0)),
            pl.BlockSpec((H, H), lambda: (0, 0)),
            pl.BlockSpec((1, H), lambda: (0, 0)),
            pl.BlockSpec((H, 128), lambda: (0, 0)),
            pl.BlockSpec((1, 128), lambda: (0, 0)),
        ],
        out_specs=pl.BlockSpec((BATCH, 128), lambda: (0, 0)),
        out_shape=jax.ShapeDtypeStruct((BATCH, 128), jnp.float32),
    )(xg, ag, s, wx, wa, bu, wpo, bpo, wlo_pad, blo_pad)


def _sc_scatter_body(y2d, srcr, dstr, ewr, out,
                     src_v, dst_v, ew_v, sdst_v, rows_v, zbuf, spmem,
                     msem, gsem, ssem):
    c = lax.axis_index("c")
    s = lax.axis_index("s")

    def zfill(i, _):
        zbuf[i, 0:16] = jnp.zeros((16,), jnp.float32)
        return 0
    lax.fori_loop(0, ZROWS, zfill, 0)

    for p in range(PASSES):
        q = c * PASSES + p
        src_off = q * N_NODES

        for r in range(ZCH):
            pltpu.sync_copy(
                zbuf, spmem.at[pl.ds(s * ROWS_PER_TILE + r * ZROWS, ZROWS)])
        plsc.subcore_barrier()

        last = CHUNKS_PER_TILE - 1
        cbase = s * CHUNKS_PER_TILE

        def issue_meta(k, b):
            base = (cbase + k) * SUB
            m = [pltpu.async_copy(srcr.at[pl.ds(base, SUB)], src_v.at[b], msem),
                 pltpu.async_copy(dstr.at[pl.ds(base, SUB)], dst_v.at[b], msem),
                 pltpu.async_copy(ewr.at[pl.ds(base, SUB)], ew_v.at[b], msem)]
            return m

        def wait_meta(b):
            for hbm, ref in ((srcr, src_v), (dstr, dst_v), (ewr, ew_v)):
                pltpu.make_async_copy(hbm.at[pl.ds(0, SUB)], ref.at[b],
                                      msem).wait()

        def add_off(b):
            for j in range(SUB):
                for l in range(8):
                    src_v[b, j, l * 16:(l + 1) * 16] = (
                        src_v[b, j, l * 16:(l + 1) * 16] + src_off)

        def issue_gather(b):
            for j in range(SUB):
                pltpu.async_copy(y2d.at[src_v.at[b, j]],
                                 rows_v.at[b, pl.ds(j * 128, 128)], gsem)

        def wait_gather(b):
            for j in range(SUB):
                pltpu.make_async_copy(y2d.at[src_v.at[b, j]],
                                      rows_v.at[b, pl.ds(j * 128, 128)],
                                      gsem).wait()

        def copy_dst(b):
            for j in range(SUB):
                for l in range(8):
                    sdst_v[b, j, l * 16:(l + 1) * 16] = (
                        dst_v[b, j, l * 16:(l + 1) * 16])

        def issue_scatter(b):
            for j in range(SUB):
                pltpu.async_copy(rows_v.at[b, pl.ds(j * 128, 128)],
                                 spmem.at[sdst_v.at[b, j]], ssem, add=True)

        def wait_scatter(b):
            for j in range(SUB):
                pltpu.make_async_copy(rows_v.at[b, pl.ds(j * 128, 128)],
                                      spmem.at[sdst_v.at[b, j]], ssem).wait()

        def scale(b):
            def scale16(g, _):
                ewv = ew_v[b, g // 8, pl.ds((g % 8) * 16, 16)]
                r0 = g * 16
                for t in range(16):
                    w = ewv[t]
                    rows_v[b, r0 + t, 0:16] = rows_v[b, r0 + t, 0:16] * w
                return 0
            lax.fori_loop(0, CHUNK // 16, scale16, 0, unroll=8)

        def process(k, b):
            @pl.when(jnp.logical_and(k >= 1, k <= last - 1))
            def _():
                wait_meta(1 - b)

            @pl.when(k <= last - 1)
            def _():
                add_off(1 - b)

            @pl.when(k >= 1)
            def _():
                wait_scatter(1 - b)

            @pl.when(k <= last - 1)
            def _():
                issue_gather(1 - b)
            wait_gather(b)

            @pl.when(k <= last - 2)
            def _():
                issue_meta(k + 2, b)
            copy_dst(b)
            scale(b)
            issue_scatter(b)

        for m in issue_meta(0, 0):
            m.wait()
        add_off(0)
        issue_gather(0)
        for m in issue_meta(1, 1):
            m.wait()

        def body2(i, _):
            process(2 * i, 0)
            process(2 * i + 1, 1)
            return 0

        lax.fori_loop(0, CHUNKS_PER_TILE // 2, body2, 0)
        wait_scatter(last % 2)
        plsc.subcore_barrier()

        for r in range(ZCH):
            row0 = s * ROWS_PER_TILE + r * ZROWS
            pltpu.sync_copy(spmem.at[pl.ds(row0, ZROWS)],
                            out.at[pl.ds(q * NP + row0, ZROWS)])


def _sc_scatter(y2d, srcr, dstr, ewr):
    mesh = plsc.VectorSubcoreMesh(core_axis_name="c", subcore_axis_name="s",
                                  num_cores=NC, num_subcores=NS)
    f = pl.kernel(
        _sc_scatter_body,
        out_type=jax.ShapeDtypeStruct((NQ * NP, QW), jnp.float32),
        mesh=mesh,
        compiler_params=pltpu.CompilerParams(use_tc_tiling_on_sc=False),
        scratch_types=[
            pltpu.VMEM((2, SUB, 128), jnp.int32),
            pltpu.VMEM((2, SUB, 128), jnp.int32),
            pltpu.VMEM((2, SUB, 128), jnp.float32),
            pltpu.VMEM((2, SUB, 128), jnp.int32),
            pltpu.VMEM((2, CHUNK, QW), jnp.float32),
            pltpu.VMEM((ZROWS, QW), jnp.float32),
            pltpu.VMEM_SHARED((NP, QW), jnp.float32),
            pltpu.SemaphoreType.DMA,
            pltpu.SemaphoreType.DMA,
            pltpu.SemaphoreType.DMA,
        ],
    )
    return f(y2d, srcr, dstr, ewr)


def _sc_gather_body(x2, aggf, idxr, xg, ag, idx_v, idx2_v, rows64, rows16, sem):
    c = lax.axis_index("c")
    s = lax.axis_index("s")
    wid = c * NS + s
    bper = BATCH // (NC * NS)
    base = wid * bper
    pltpu.sync_copy(idxr.at[pl.ds(base, bper)], idx_v)
    pltpu.async_copy(x2.at[idx_v], rows64, sem).wait()
    pltpu.sync_copy(rows64, xg.at[pl.ds(base, bper)])
    for q in range(NQ):
        for l in range(bper // 16):
            idx2_v[l * 16:(l + 1) * 16] = (idx_v[l * 16:(l + 1) * 16]
                                           + q * NP)
        pltpu.async_copy(aggf.at[idx2_v], rows16, sem).wait()
        pltpu.sync_copy(rows16, ag.at[q, pl.ds(base, bper)])


def _sc_gather(x2, aggf, idx):
    bper = BATCH // (NC * NS)
    mesh = plsc.VectorSubcoreMesh(core_axis_name="c", subcore_axis_name="s",
                                  num_cores=NC, num_subcores=NS)
    f = pl.kernel(
        _sc_gather_body,
        compiler_params=pltpu.CompilerParams(use_tc_tiling_on_sc=False),
        out_type=[
            jax.ShapeDtypeStruct((BATCH, H), jnp.float32),
            jax.ShapeDtypeStruct((NQ, BATCH, QW), jnp.float32),
        ],
        mesh=mesh,
        scratch_types=[
            pltpu.VMEM((bper,), jnp.int32),
            pltpu.VMEM((bper,), jnp.int32),
            pltpu.VMEM((bper, H), jnp.float32),
            pltpu.VMEM((bper, QW), jnp.float32),
            pltpu.SemaphoreType.DMA,
        ],
    )
    return f(x2, aggf, idx)


def kernel(node_features, edge_weights, lstm_kernel, lstm_recurrent, lstm_bias,
           c1_prep_W, c1_prep_b, c1_upd_W, c1_upd_b,
           c2_prep_W, c2_prep_b, c2_upd_W, c2_upd_b,
           post_W, post_b, logits_W, logits_b,
           edges, input_node_indices):
    pad = EDGES_PAD - N_EDGES
    src = jnp.pad(edges[1], (0, pad)).reshape(EDGES_PAD // 128, 128)
    dst = jnp.pad(edges[0], (0, pad)).reshape(EDGES_PAD // 128, 128)
    ewp = jnp.pad(edge_weights, (0, pad)).reshape(EDGES_PAD // 128, 128)

    wcat = jnp.concatenate([lstm_kernel, lstm_recurrent], axis=0)
    b4 = lstm_bias.reshape(1, 4 * U)
    wp1 = c1_prep_W * BN_INV
    bp1 = c1_prep_b.reshape(1, H)
    u1 = c1_upd_W * BN_INV
    wx1 = u1[:U]
    wa1 = u1[U:].reshape(NQ, QW, H)
    bu1 = c1_upd_b.reshape(1, H)
    wp2 = c2_prep_W * BN_INV
    bp2 = c2_prep_b.reshape(1, H)
    u2 = c2_upd_W * BN_INV
    wx2 = u2[:H]
    wa2 = u2[H:].reshape(NQ, QW, H)
    bu2 = c2_upd_b.reshape(1, H)
    wpo = post_W * BN_INV
    bpo = post_b.reshape(1, H)
    wlo = jnp.pad(logits_W, ((0, 0), (0, 128 - NUM_CLASSES)))
    blo = jnp.pad(logits_b, (0, 128 - NUM_CLASSES)).reshape(1, 128)

    s_sum = _ew_sum(ewp)
    x1, y1 = _lstm_prep(node_features, wcat, b4, wp1, bp1)
    agg1 = _sc_scatter(y1.reshape(NQ * N_NODES, QW), src, dst, ewp)
    x2, y2 = _upd_prep(x1, agg1.reshape(NQ, NP, QW), s_sum,
                       wx1, wa1, bu1, wp2, bp2)
    agg2 = _sc_scatter(y2.reshape(NQ * N_NODES, QW), src, dst, ewp)
    xg, ag = _sc_gather(x2, agg2, input_node_indices)
    out = _head(xg, ag, s_sum, wx2, wa2, bu2, wpo, bpo, wlo, blo)
    return out[:, :NUM_CLASSES]


# speedup vs baseline: 5.8718x; 1.0010x over previous
import math

import jax
import jax.numpy as jnp
from jax import lax
from jax.experimental import pallas as pl
from jax.experimental.pallas import tpu as pltpu
from jax.experimental.pallas import tpu_sc as plsc

N_NODES = 50000
N_EDGES = 800000
T = 8
F = 64
U = 64
H = 64
NUM_CLASSES = 40
BATCH = 1024
BN_INV = (1.0 + 1e-3) ** -0.5

NC = 2
NS = 16
QW = 16
NQ = 4
PASSES = NQ // NC

CHUNK = 1024
SUB = CHUNK // 128
CHUNKS_PER_TILE = 2 * math.ceil(N_EDGES / (NS * CHUNK * 2))
EDGES_PAD = NS * CHUNK * CHUNKS_PER_TILE
NP = 51200
ROWS_PER_TILE = NP // NS
ZCH = 5
ZROWS = ROWS_PER_TILE // ZCH


def _sum_body(x_ref, o_ref):
    o_ref[0, 0] = jnp.sum(x_ref[...])


def _ew_sum(ew_pad2d):
    return pl.pallas_call(
        _sum_body,
        out_specs=pl.BlockSpec(memory_space=pltpu.SMEM),
        out_shape=jax.ShapeDtypeStruct((1, 1), jnp.float32),
    )(ew_pad2d)


def _gelu(x):
    return 0.5 * x * (1.0 + lax.erf(x * (2.0 ** -0.5)))


def _lstm_body(x_ref, wc_ref, b_ref, wp_ref, bp_ref, x1_ref, y1_ref):
    xb = x_ref[...]
    bn = xb.shape[0]
    wc = wc_ref[...]
    b = b_ref[...]
    h = jnp.zeros((bn, U), jnp.float32)
    c = jnp.zeros((bn, U), jnp.float32)
    for t in range(T):
        xt = xb[:, t, :]
        z = jnp.dot(jnp.concatenate([xt, h], axis=1), wc,
                    preferred_element_type=jnp.float32) + b
        zi = z[:, 0:U]
        zf = z[:, U:2 * U]
        zg = z[:, 2 * U:3 * U]
        zo = z[:, 3 * U:4 * U]
        c = jax.nn.sigmoid(zf) * c + jax.nn.sigmoid(zi) * jnp.tanh(zg)
        h = jax.nn.sigmoid(zo) * jnp.tanh(c)
    x1_ref[...] = h
    y = _gelu(jnp.dot(h, wp_ref[...], preferred_element_type=jnp.float32)
              + bp_ref[...])
    for q in range(NQ):
        y1_ref[q] = y[:, q * QW:(q + 1) * QW]


def _lstm_prep(nf, wcat, b4, wp, bp, bn=2000):
    grid = N_NODES // bn
    return pl.pallas_call(
        _lstm_body,
        grid=(grid,),
        in_specs=[
            pl.BlockSpec((bn, T, F), lambda i: (i, 0, 0)),
            pl.BlockSpec((F + U, 4 * U), lambda i: (0, 0)),
            pl.BlockSpec((1, 4 * U), lambda i: (0, 0)),
            pl.BlockSpec((U, H), lambda i: (0, 0)),
            pl.BlockSpec((1, H), lambda i: (0, 0)),
        ],
        out_specs=[
            pl.BlockSpec((bn, U), lambda i: (i, 0)),
            pl.BlockSpec((NQ, bn, QW), lambda i: (0, i, 0)),
        ],
        out_shape=[
            jax.ShapeDtypeStruct((N_NODES, U), jnp.float32),
            jax.ShapeDtypeStruct((NQ, N_NODES, QW), jnp.float32),
        ],
    )(nf, wcat, b4, wp, bp)


def _upd_body(x_ref, agg_ref, s_ref, wx_ref, wa_ref, bu_ref,
              wp_ref, bp_ref, x2_ref, y2_ref):
    x = x_ref[...]
    inv_s = 1.0 / s_ref[0, 0]
    a = jnp.dot(agg_ref[0], wa_ref[0], preferred_element_type=jnp.float32)
    for q in range(1, NQ):
        a = a + jnp.dot(agg_ref[q], wa_ref[q], preferred_element_type=jnp.float32)
    u = (jnp.dot(x, wx_ref[...], preferred_element_type=jnp.float32)
         + a * inv_s + bu_ref[...])
    o = _gelu(u)
    o = o * lax.rsqrt(jnp.maximum(jnp.sum(o * o, axis=-1, keepdims=True), 1e-12))
    x2_ref[...] = o
    y = _gelu(jnp.dot(o, wp_ref[...], preferred_element_type=jnp.float32)
              + bp_ref[...])
    for q in range(NQ):
        y2_ref[q] = y[:, q * QW:(q + 1) * QW]


def _upd_prep(x1, agg, s, wx, wa, bu, wp, bp, bn=2000):
    grid = N_NODES // bn
    return pl.pallas_call(
        _upd_body,
        grid=(grid,),
        in_specs=[
            pl.BlockSpec((bn, U), lambda i: (i, 0)),
            pl.BlockSpec((NQ, bn, QW), lambda i: (0, i, 0)),
            pl.BlockSpec(memory_space=pltpu.SMEM),
            pl.BlockSpec((U, H), lambda i: (0, 0)),
            pl.BlockSpec((NQ, QW, H), lambda i: (0, 0, 0)),
            pl.BlockSpec((1, H), lambda i: (0, 0)),
            pl.BlockSpec((H, H), lambda i: (0, 0)),
            pl.BlockSpec((1, H), lambda i: (0, 0)),
        ],
        out_specs=[
            pl.BlockSpec((bn, H), lambda i: (i, 0)),
            pl.BlockSpec((NQ, bn, QW), lambda i: (0, i, 0)),
        ],
        out_shape=[
            jax.ShapeDtypeStruct((N_NODES, H), jnp.float32),
            jax.ShapeDtypeStruct((NQ, N_NODES, QW), jnp.float32),
        ],
    )(x1, agg, s, wx, wa, bu, wp, bp)


def _head_body(xg_ref, ag_ref, s_ref, wx_ref, wa_ref,
               bu_ref, wpo_ref, bpo_ref, wlo_ref, blo_ref, o_ref):
    inv_s = 1.0 / s_ref[0, 0]
    a = jnp.dot(ag_ref[0], wa_ref[0], preferred_element_type=jnp.float32)
    for q in range(1, NQ):
        a = a + jnp.dot(ag_ref[q], wa_ref[q], preferred_element_type=jnp.float32)
    u = (jnp.dot(xg_ref[...], wx_ref[...], preferred_element_type=jnp.float32)
         + a * inv_s + bu_ref[...])
    o = _gelu(u)
    x3 = o * lax.rsqrt(jnp.maximum(jnp.sum(o * o, axis=-1, keepdims=True), 1e-12))
    x4 = _gelu(jnp.dot(x3, wpo_ref[...], preferred_element_type=jnp.float32)
               + bpo_ref[...])
    o_ref[...] = (jnp.dot(x4, wlo_ref[...], preferred_element_type=jnp.float32)
                  + blo_ref[...])


def _head(xg, ag, s, wx, wa, bu, wpo, bpo, wlo_pad, blo_pad):
    return pl.pallas_call(
        _head_body,
        in_specs=[
            pl.BlockSpec((BATCH, H), lambda: (0, 0)),
            pl.BlockSpec((NQ, BATCH, QW), lambda: (0, 0, 0)),
            pl.BlockSpec(memory_space=pltpu.SMEM),
            pl.BlockSpec((H, H), lambda: (0, 0)),
            pl.BlockSpec((NQ, QW, H), lambda: (0, 0, 0)),
            pl.BlockSpec((1, H), lambda: (0, 0)),
            pl.BlockSpec((H, H), lambda: (0, 0)),
            pl.BlockSpec((1, H), lambda: (0, 0)),
            pl.BlockSpec((H, 128), lambda: (0, 0)),
            pl.BlockSpec((1, 128), lambda: (0, 0)),
        ],
        out_specs=pl.BlockSpec((BATCH, 128), lambda: (0, 0)),
        out_shape=jax.ShapeDtypeStruct((BATCH, 128), jnp.float32),
    )(xg, ag, s, wx, wa, bu, wpo, bpo, wlo_pad, blo_pad)


def _sc_scatter_body(y2d, srcr, dstr, ewr, out,
                     src_v, dst_v, ew_v, src_f, sdst_v, rows_v, zbuf, spmem,
                     msem, gsem, ssem):
    c = lax.axis_index("c")
    s = lax.axis_index("s")

    def zfill(i, _):
        zbuf[i, 0:16] = jnp.zeros((16,), jnp.float32)
        return 0
    lax.fori_loop(0, ZROWS, zfill, 0)

    for p in range(PASSES):
        q = c * PASSES + p
        src_off = q * N_NODES

        for r in range(ZCH):
            pltpu.sync_copy(
                zbuf, spmem.at[pl.ds(s * ROWS_PER_TILE + r * ZROWS, ZROWS)])
        plsc.subcore_barrier()

        last = CHUNKS_PER_TILE - 1
        cbase = s * CHUNKS_PER_TILE

        def issue_meta(k, b):
            base = (cbase + k) * SUB
            m = [pltpu.async_copy(srcr.at[pl.ds(base, SUB)], src_v.at[b], msem),
                 pltpu.async_copy(dstr.at[pl.ds(base, SUB)], dst_v.at[b], msem),
                 pltpu.async_copy(ewr.at[pl.ds(base, SUB)], ew_v.at[b], msem)]
            return m

        def wait_meta(b):
            for hbm, ref in ((srcr, src_v), (dstr, dst_v), (ewr, ew_v)):
                pltpu.make_async_copy(hbm.at[pl.ds(0, SUB)], ref.at[b],
                                      msem).wait()

        def add_off(b):
            for j in range(SUB):
                for l in range(8):
                    src_f[b, pl.ds(j * 128 + l * 16, 16)] = (
                        src_v[b, j, l * 16:(l + 1) * 16] + src_off)

        def issue_gather(b):
            pltpu.async_copy(y2d.at[src_f.at[b]], rows_v.at[b], gsem)

        def wait_gather(b):
            pltpu.make_async_copy(y2d.at[src_f.at[b]], rows_v.at[b],
                                  gsem).wait()

        def copy_dst(b):
            for j in range(SUB):
                for l in range(8):
                    sdst_v[b, pl.ds(j * 128 + l * 16, 16)] = (
                        dst_v[b, j, l * 16:(l + 1) * 16])

        def issue_scatter(b):
            pltpu.async_copy(rows_v.at[b], spmem.at[sdst_v.at[b]], ssem,
                             add=True)

        def wait_scatter(b):
            pltpu.make_async_copy(rows_v.at[b], spmem.at[sdst_v.at[b]],
                                  ssem).wait()

        def scale(b):
            def scale16(g, _):
                ewv = ew_v[b, g // 8, pl.ds((g % 8) * 16, 16)]
                r0 = g * 16
                for t in range(16):
                    w = ewv[t]
                    rows_v[b, r0 + t, 0:16] = rows_v[b, r0 + t, 0:16] * w
                return 0
            lax.fori_loop(0, CHUNK // 16, scale16, 0, unroll=8)

        def process(k, b):
            @pl.when(jnp.logical_and(k >= 1, k <= last - 1))
            def _():
                wait_meta(1 - b)

            @pl.when(k <= last - 1)
            def _():
                add_off(1 - b)

            @pl.when(k >= 1)
            def _():
                wait_scatter(1 - b)

            @pl.when(k <= last - 1)
            def _():
                issue_gather(1 - b)
            wait_gather(b)

            @pl.when(k <= last - 2)
            def _():
                issue_meta(k + 2, b)
            copy_dst(b)
            scale(b)
            issue_scatter(b)

        for m in issue_meta(0, 0):
            m.wait()
        add_off(0)
        issue_gather(0)
        for m in issue_meta(1, 1):
            m.wait()

        def body2(i, _):
            process(2 * i, 0)
            process(2 * i + 1, 1)
            return 0

        lax.fori_loop(0, CHUNKS_PER_TILE // 2, body2, 0)
        wait_scatter(last % 2)
        plsc.subcore_barrier()

        for r in range(ZCH):
            row0 = s * ROWS_PER_TILE + r * ZROWS
            pltpu.sync_copy(spmem.at[pl.ds(row0, ZROWS)],
                            out.at[pl.ds(q * NP + row0, ZROWS)])


def _sc_scatter(y2d, srcr, dstr, ewr):
    mesh = plsc.VectorSubcoreMesh(core_axis_name="c", subcore_axis_name="s",
                                  num_cores=NC, num_subcores=NS)
    f = pl.kernel(
        _sc_scatter_body,
        out_type=jax.ShapeDtypeStruct((NQ * NP, QW), jnp.float32),
        mesh=mesh,
        compiler_params=pltpu.CompilerParams(use_tc_tiling_on_sc=False),
        scratch_types=[
            pltpu.VMEM((2, SUB, 128), jnp.int32),
            pltpu.VMEM((2, SUB, 128), jnp.int32),
            pltpu.VMEM((2, SUB, 128), jnp.float32),
            pltpu.VMEM((2, CHUNK), jnp.int32),
            pltpu.VMEM((2, CHUNK), jnp.int32),
            pltpu.VMEM((2, CHUNK, QW), jnp.float32),
            pltpu.VMEM((ZROWS, QW), jnp.float32),
            pltpu.VMEM_SHARED((NP, QW), jnp.float32),
            pltpu.SemaphoreType.DMA,
            pltpu.SemaphoreType.DMA,
            pltpu.SemaphoreType.DMA,
        ],
    )
    return f(y2d, srcr, dstr, ewr)


def _sc_gather_body(x2, aggf, idxr, xg, ag, idx_v, idx2_v, rows64, rows16, sem):
    c = lax.axis_index("c")
    s = lax.axis_index("s")
    wid = c * NS + s
    bper = BATCH // (NC * NS)
    base = wid * bper
    pltpu.sync_copy(idxr.at[pl.ds(base, bper)], idx_v)
    pltpu.async_copy(x2.at[idx_v], rows64, sem).wait()
    pltpu.sync_copy(rows64, xg.at[pl.ds(base, bper)])
    for q in range(NQ):
        for l in range(bper // 16):
            idx2_v[l * 16:(l + 1) * 16] = (idx_v[l * 16:(l + 1) * 16]
                                           + q * NP)
        pltpu.async_copy(aggf.at[idx2_v], rows16, sem).wait()
        pltpu.sync_copy(rows16, ag.at[q, pl.ds(base, bper)])


def _sc_gather(x2, aggf, idx):
    bper = BATCH // (NC * NS)
    mesh = plsc.VectorSubcoreMesh(core_axis_name="c", subcore_axis_name="s",
                                  num_cores=NC, num_subcores=NS)
    f = pl.kernel(
        _sc_gather_body,
        compiler_params=pltpu.CompilerParams(use_tc_tiling_on_sc=False),
        out_type=[
            jax.ShapeDtypeStruct((BATCH, H), jnp.float32),
            jax.ShapeDtypeStruct((NQ, BATCH, QW), jnp.float32),
        ],
        mesh=mesh,
        scratch_types=[
            pltpu.VMEM((bper,), jnp.int32),
            pltpu.VMEM((bper,), jnp.int32),
            pltpu.VMEM((bper, H), jnp.float32),
            pltpu.VMEM((bper, QW), jnp.float32),
            pltpu.SemaphoreType.DMA,
        ],
    )
    return f(x2, aggf, idx)


def kernel(node_features, edge_weights, lstm_kernel, lstm_recurrent, lstm_bias,
           c1_prep_W, c1_prep_b, c1_upd_W, c1_upd_b,
           c2_prep_W, c2_prep_b, c2_upd_W, c2_upd_b,
           post_W, post_b, logits_W, logits_b,
           edges, input_node_indices):
    pad = EDGES_PAD - N_EDGES
    src = jnp.pad(edges[1], (0, pad)).reshape(EDGES_PAD // 128, 128)
    dst = jnp.pad(edges[0], (0, pad)).reshape(EDGES_PAD // 128, 128)
    ewp = jnp.pad(edge_weights, (0, pad)).reshape(EDGES_PAD // 128, 128)

    wcat = jnp.concatenate([lstm_kernel, lstm_recurrent], axis=0)
    b4 = lstm_bias.reshape(1, 4 * U)
    wp1 = c1_prep_W * BN_INV
    bp1 = c1_prep_b.reshape(1, H)
    u1 = c1_upd_W * BN_INV
    wx1 = u1[:U]
    wa1 = u1[U:].reshape(NQ, QW, H)
    bu1 = c1_upd_b.reshape(1, H)
    wp2 = c2_prep_W * BN_INV
    bp2 = c2_prep_b.reshape(1, H)
    u2 = c2_upd_W * BN_INV
    wx2 = u2[:H]
    wa2 = u2[H:].reshape(NQ, QW, H)
    bu2 = c2_upd_b.reshape(1, H)
    wpo = post_W * BN_INV
    bpo = post_b.reshape(1, H)
    wlo = jnp.pad(logits_W, ((0, 0), (0, 128 - NUM_CLASSES)))
    blo = jnp.pad(logits_b, (0, 128 - NUM_CLASSES)).reshape(1, 128)

    s_sum = _ew_sum(ewp)
    x1, y1 = _lstm_prep(node_features, wcat, b4, wp1, bp1)
    agg1 = _sc_scatter(y1.reshape(NQ * N_NODES, QW), src, dst, ewp)
    x2, y2 = _upd_prep(x1, agg1.reshape(NQ, NP, QW), s_sum,
                       wx1, wa1, bu1, wp2, bp2)
    agg2 = _sc_scatter(y2.reshape(NQ * N_NODES, QW), src, dst, ewp)
    xg, ag = _sc_gather(x2, agg2, input_node_indices)
    out = _head(xg, ag, s_sum, wx2, wa2, bu2, wpo, bpo, wlo, blo)
    return out[:, :NUM_CLASSES]
